# baseline (speedup 1.0000x reference)
import functools

import jax
import jax.numpy as jnp
from jax import lax
from jax.experimental import pallas as pl
from jax.experimental.pallas import tpu as pltpu
from jax.experimental.pallas import tpu_sc as plsc

N = 10000
E = 320000
D = 128
NC = 2
NS = 16
NW = NC * NS
C = 80
NCHUNKS = E // C
NK = NCHUNKS // NW
ROWS_PER_TILE = N // NS
RR = 2
IR = 8


def _scale_chunk(raw, scaled, vv, b, j):
    hi_mask = jnp.full((16,), -65536, jnp.int32)

    @plsc.parallel_loop(0, C // 16, unroll=5)
    def _group(g):
        v16 = vv[j, pl.ds(g * 16, 16)]
        for t in range(16):
            vt = v16[t]
            e = g * 16 + t
            ps = [raw[b, e, pl.ds(jj * 16, 16)] for jj in range(4)]
            los = [lax.bitcast_convert_type(lax.shift_left(p, 16),
                                            jnp.float32) for p in ps]
            his = [lax.bitcast_convert_type(jnp.bitwise_and(p, hi_mask),
                                            jnp.float32) for p in ps]
            for jj in range(4):
                scaled[b, e, pl.ds(jj * 32, 16)] = los[jj] * vt
                scaled[b, e, pl.ds(jj * 32 + 16, 16)] = his[jj] * vt


def _sc_agg_body(x_hbm, row_hbm, col_hbm, val_hbm, out_hbm,
                 ci, vv, raw, scaled, acc, gi_sem, g_sem, s_sem):
    cid = lax.axis_index("c")
    sid = lax.axis_index("s")
    wid = sid * NC + cid

    zero16 = jnp.zeros((16,), jnp.float32)

    def _zero_rowsbuf(i, carry):
        for jj in range(8):
            scaled[0, i, pl.ds(jj * 16, 16)] = zero16
        return carry

    lax.fori_loop(0, C, _zero_rowsbuf, 0)
    for k in range(7):
        pltpu.sync_copy(scaled.at[0],
                        acc.at[pl.ds(sid * ROWS_PER_TILE + k * C, C)])
    pltpu.sync_copy(scaled.at[0, pl.ds(0, 65)],
                    acc.at[pl.ds(sid * ROWS_PER_TILE + 7 * C, 65)])
    plsc.subcore_barrier()

    def _chunk_base(k):
        return (wid * NK + k) * C

    def _issue_idx(k):
        slot = lax.rem(k, IR)
        base = _chunk_base(k)
        pltpu.async_copy(row_hbm.at[pl.ds(base, C)], ci.at[slot, 0],
                         gi_sem.at[slot])
        pltpu.async_copy(col_hbm.at[pl.ds(base, C)], ci.at[slot, 1],
                         gi_sem.at[slot])
        pltpu.async_copy(val_hbm.at[pl.ds(base, C)], vv.at[slot],
                         gi_sem.at[slot])

    def _wait_idx(k):
        slot = lax.rem(k, IR)
        base = _chunk_base(k)
        pltpu.make_async_copy(row_hbm.at[pl.ds(base, C)], ci.at[slot, 0],
                              gi_sem.at[slot]).wait()
        pltpu.make_async_copy(col_hbm.at[pl.ds(base, C)], ci.at[slot, 1],
                              gi_sem.at[slot]).wait()
        pltpu.make_async_copy(val_hbm.at[pl.ds(base, C)], vv.at[slot],
                              gi_sem.at[slot]).wait()

    def _issue_gather(k):
        slot = lax.rem(k, RR)
        islot = lax.rem(k, IR)
        pltpu.async_copy(x_hbm.at[ci.at[islot, 1]], raw.at[slot],
                         g_sem.at[slot])

    def _wait_gather(k):
        slot = lax.rem(k, RR)
        islot = lax.rem(k, IR)
        pltpu.make_async_copy(x_hbm.at[ci.at[islot, 1]], raw.at[slot],
                              g_sem.at[slot]).wait()

    def _issue_scatter(k):
        slot = lax.rem(k, RR)
        islot = lax.rem(k, IR)
        pltpu.async_copy(scaled.at[slot], acc.at[ci.at[islot, 0]],
                         s_sem.at[slot], add=True)

    def _wait_scatter(k):
        slot = lax.rem(k, RR)
        islot = lax.rem(k, IR)
        pltpu.make_async_copy(scaled.at[slot], acc.at[ci.at[islot, 0]],
                              s_sem.at[slot]).wait()

    for p in range(3):
        _issue_idx(p)
    for p in range(2):
        _wait_idx(p)
        _issue_gather(p)

    def _loop_body(k, carry):
        _wait_gather(k)

        @pl.when(k >= 2)
        def _():
            _wait_scatter(k - 2)

        _scale_chunk(raw, scaled, vv, lax.rem(k, RR), lax.rem(k, IR))

        @pl.when(k + 2 < NK)
        def _():
            _wait_idx(k + 2)
            _issue_gather(k + 2)

        _issue_scatter(k)

        @pl.when(k + 3 < NK)
        def _():
            _issue_idx(k + 3)

        return carry

    lax.fori_loop(0, NK, _loop_body, 0)
    _wait_scatter(NK - 2)
    _wait_scatter(NK - 1)

    plsc.subcore_barrier()

    @pl.when(sid == 0)
    def _():
        pltpu.sync_copy(acc, out_hbm.at[cid])


_sc_agg = pl.kernel(
    _sc_agg_body,
    out_type=jax.ShapeDtypeStruct((NC, N, D), jnp.float32),
    mesh=plsc.VectorSubcoreMesh(core_axis_name="c", subcore_axis_name="s"),
    compiler_params=pltpu.CompilerParams(use_tc_tiling_on_sc=False),
    scratch_types=[
        pltpu.VMEM((IR, 2, C), jnp.int32),
        pltpu.VMEM((IR, C), jnp.float32),
        pltpu.VMEM((RR, C, D // 2), jnp.int32),
        pltpu.VMEM((RR, C, D), jnp.float32),
        pltpu.VMEM_SHARED((N, D), jnp.float32),
        pltpu.SemaphoreType.DMA((IR,)),
        pltpu.SemaphoreType.DMA((RR,)),
        pltpu.SemaphoreType.DMA((RR,)),
    ],
)


def _tc_combine_body(p_ref, w_ref, b_ref, o_ref):
    s = p_ref[0] + p_ref[1]
    o_ref[...] = (
        jnp.dot(s, w_ref[...], preferred_element_type=jnp.float32) + b_ref[...]
    )


_RB = 1000


@jax.jit
def _tc_combine(partials, Wt, b2):
    return pl.pallas_call(
        _tc_combine_body,
        grid=(N // _RB,),
        in_specs=[
            pl.BlockSpec((NC, _RB, D), lambda i: (0, i, 0)),
            pl.BlockSpec((D, D), lambda i: (0, 0)),
            pl.BlockSpec((1, D), lambda i: (0, 0)),
        ],
        out_specs=pl.BlockSpec((_RB, D), lambda i: (i, 0)),
        out_shape=jax.ShapeDtypeStruct((N, D), jnp.float32),
    )(partials, Wt, b2)


_PERM = [
    (n // 32) * 32 + 2 * (n % 16) + (n % 32) // 16 for n in range(D)
]


def kernel(x, adj_indices, adj_values, W, b):
    adj = adj_indices.astype(jnp.int32)
    xh = x.astype(jnp.bfloat16)
    xi = lax.bitcast_convert_type(xh.reshape(N, D // 2, 2), jnp.int32)
    partials = _sc_agg(xi, adj[0], adj[1], adj_values)
    Wt_perm = W.T[jnp.array(_PERM, dtype=jnp.int32)]
    return _tc_combine(partials, Wt_perm, b.reshape(1, D))


# speedup vs baseline: 1.0794x; 1.0794x over previous
import functools

import jax
import jax.numpy as jnp
from jax import lax
from jax.experimental import pallas as pl
from jax.experimental.pallas import tpu as pltpu
from jax.experimental.pallas import tpu_sc as plsc

N = 10000
E = 320000
D = 128
NC = 2
NS = 16
NW = NC * NS
C = 80
NCHUNKS = E // C
NK = NCHUNKS // NW
ROWS_PER_TILE = N // NS
RR = 2
IR = 8


def _scale_chunk(raw, scaled, vv, b, j):
    @plsc.parallel_loop(0, C // 16, unroll=5)
    def _group(g):
        v16 = vv[j, pl.ds(g * 16, 16)]
        for t in range(16):
            vt = v16[t]
            e = g * 16 + t
            for jj in range(8):
                sl = pl.ds(jj * 16, 16)
                scaled[b, e, sl] = raw[b, e, sl] * vt


def _sc_agg_body(x_hbm, row_hbm, col_hbm, val_hbm, out_hbm,
                 ci, vv, raw, scaled, acc, gi_sem, g_sem, s_sem):
    cid = lax.axis_index("c")
    sid = lax.axis_index("s")
    wid = sid * NC + cid

    def _chunk_base(k):
        return (wid * NK + k) * C

    def _issue_idx(k):
        slot = lax.rem(k, IR)
        base = _chunk_base(k)
        pltpu.async_copy(row_hbm.at[pl.ds(base, C)], ci.at[slot, 0],
                         gi_sem.at[slot])
        pltpu.async_copy(col_hbm.at[pl.ds(base, C)], ci.at[slot, 1],
                         gi_sem.at[slot])
        pltpu.async_copy(val_hbm.at[pl.ds(base, C)], vv.at[slot],
                         gi_sem.at[slot])

    def _wait_idx(k):
        slot = lax.rem(k, IR)
        base = _chunk_base(k)
        pltpu.make_async_copy(row_hbm.at[pl.ds(base, C)], ci.at[slot, 0],
                              gi_sem.at[slot]).wait()
        pltpu.make_async_copy(col_hbm.at[pl.ds(base, C)], ci.at[slot, 1],
                              gi_sem.at[slot]).wait()
        pltpu.make_async_copy(val_hbm.at[pl.ds(base, C)], vv.at[slot],
                              gi_sem.at[slot]).wait()

    def _issue_gather(k):
        slot = lax.rem(k, RR)
        islot = lax.rem(k, IR)
        pltpu.async_copy(x_hbm.at[ci.at[islot, 1]], raw.at[slot],
                         g_sem.at[slot])

    def _wait_gather(k):
        slot = lax.rem(k, RR)
        islot = lax.rem(k, IR)
        pltpu.make_async_copy(x_hbm.at[ci.at[islot, 1]], raw.at[slot],
                              g_sem.at[slot]).wait()

    def _issue_scatter(k):
        slot = lax.rem(k, RR)
        islot = lax.rem(k, IR)
        pltpu.async_copy(scaled.at[slot], acc.at[ci.at[islot, 0]],
                         s_sem.at[slot], add=True)

    def _wait_scatter(k):
        slot = lax.rem(k, RR)
        islot = lax.rem(k, IR)
        pltpu.make_async_copy(scaled.at[slot], acc.at[ci.at[islot, 0]],
                              s_sem.at[slot]).wait()

    for p in range(3):
        _issue_idx(p)

    zero16 = jnp.zeros((16,), jnp.float32)

    def _zero_rowsbuf(i, carry):
        for jj in range(8):
            raw[0, i, pl.ds(jj * 16, 16)] = zero16
        return carry

    lax.fori_loop(0, C, _zero_rowsbuf, 0)

    def _zero_copies():
        for k in range(7):
            yield pltpu.make_async_copy(
                raw.at[0],
                acc.at[pl.ds(sid * ROWS_PER_TILE + k * C, C)],
                s_sem.at[0])
        yield pltpu.make_async_copy(
            raw.at[0, pl.ds(0, 65)],
            acc.at[pl.ds(sid * ROWS_PER_TILE + 7 * C, 65)],
            s_sem.at[0])

    for c in _zero_copies():
        c.start()
    for c in _zero_copies():
        c.wait()
    plsc.subcore_barrier()

    for p in range(2):
        _wait_idx(p)
        _issue_gather(p)

    def _loop_body(k, carry):
        _wait_gather(k)

        @pl.when(k >= 2)
        def _():
            _wait_scatter(k - 2)

        _scale_chunk(raw, scaled, vv, lax.rem(k, RR), lax.rem(k, IR))

        @pl.when(k + 2 < NK)
        def _():
            _wait_idx(k + 2)
            _issue_gather(k + 2)

        _issue_scatter(k)

        @pl.when(k + 3 < NK)
        def _():
            _issue_idx(k + 3)

        return carry

    lax.fori_loop(0, NK, _loop_body, 0)
    _wait_scatter(NK - 2)
    _wait_scatter(NK - 1)

    plsc.subcore_barrier()

    @pl.when(sid == 0)
    def _():
        pltpu.sync_copy(acc, out_hbm.at[cid])


_sc_agg = pl.kernel(
    _sc_agg_body,
    out_type=jax.ShapeDtypeStruct((NC, N, D), jnp.float32),
    mesh=plsc.VectorSubcoreMesh(core_axis_name="c", subcore_axis_name="s"),
    scratch_types=[
        pltpu.VMEM((IR, 2, C), jnp.int32),
        pltpu.VMEM((IR, C), jnp.float32),
        pltpu.VMEM((RR, C, D), jnp.float32),
        pltpu.VMEM((RR, C, D), jnp.float32),
        pltpu.VMEM_SHARED((N, D), jnp.float32),
        pltpu.SemaphoreType.DMA((IR,)),
        pltpu.SemaphoreType.DMA((RR,)),
        pltpu.SemaphoreType.DMA((RR,)),
    ],
)


def _tc_combine_body(p_ref, w_ref, b_ref, o_ref):
    s = p_ref[0] + p_ref[1]
    o_ref[...] = (
        jnp.dot(s, w_ref[...], preferred_element_type=jnp.float32) + b_ref[...]
    )


_RB = 1000


@jax.jit
def _tc_combine(partials, Wt, b2):
    return pl.pallas_call(
        _tc_combine_body,
        grid=(N // _RB,),
        in_specs=[
            pl.BlockSpec((NC, _RB, D), lambda i: (0, i, 0)),
            pl.BlockSpec((D, D), lambda i: (0, 0)),
            pl.BlockSpec((1, D), lambda i: (0, 0)),
        ],
        out_specs=pl.BlockSpec((_RB, D), lambda i: (i, 0)),
        out_shape=jax.ShapeDtypeStruct((N, D), jnp.float32),
    )(partials, Wt, b2)


def kernel(x, adj_indices, adj_values, W, b):
    adj = adj_indices.astype(jnp.int32)
    partials = _sc_agg(x, adj[0], adj[1], adj_values)
    return _tc_combine(partials, W.T, b.reshape(1, D))
